# baseline (speedup 1.0000x reference)
import jax
import jax.numpy as jnp
from jax import lax
from jax.experimental import pallas as pl
from jax.experimental.pallas import tpu as pltpu
from jax.experimental.pallas import tpu_sc as plsc

N_UP = 2500
F = 128
K = 32
N_PAD = 10240
L = 16

RG = 8
FC = 4
RPG = N_PAD // RG
SUB = 10
RPS = RPG // SUB
NBLK = RPS // L
FCW = F // FC


def _body(feat_hbm, wt_hbm, nt_hbm, out_hbm, feat_c,
          idx0, idx1, w0, w1, ob0, ob1, sin0, sin1, sout0, sout1):
    wid = lax.axis_index("s") * 2 + lax.axis_index("c")
    rg = wid // FC
    fc = wid % FC
    row0g = rg * RPG
    col0 = fc * FCW
    feat_cp = pltpu.async_copy(feat_hbm.at[pl.ds(col0, FCW)], feat_c, sout0)

    idx_b = (idx0, idx1)
    w_b = (w0, w1)
    out_b = (ob0, ob1)
    sin = (sin0, sin1)
    sout = (sout0, sout1)

    def stage(s, p):
        row0 = row0g + s * RPS
        return (pltpu.async_copy(nt_hbm.at[:, pl.ds(row0, RPS)],
                                 idx_b[p], sin[p]),
                pltpu.async_copy(wt_hbm.at[:, pl.ds(row0, RPS)],
                                 w_b[p], sin[p]))

    pending_in = stage(0, 0)
    feat_cp.wait()
    pending_out = [None, None]

    for s in range(SUB):
        p = s % 2
        row0 = row0g + s * RPS
        for cp in pending_in:
            cp.wait()
        if s + 1 < SUB:
            pending_in = stage(s + 1, (s + 1) % 2)
        if pending_out[p] is not None:
            pending_out[p].wait()
        idx_c = idx_b[p]
        w_c = w_b[p]
        out_buf = out_b[p]

        def block(b, _):
            rr = b * L
            for half in range(2):
                def kbody(k, accs):
                    vk = idx_c[k, pl.ds(rr, L)]
                    m = vk < N_UP
                    vkc = jnp.where(m, vk, 0)
                    wk = jnp.where(m, w_c[k, pl.ds(rr, L)], 0.0)
                    new = []
                    for f in range(L):
                        col = jnp.full((L,), half * L + f, jnp.int32)
                        g = plsc.load_gather(feat_c, [col, vkc])
                        new.append(accs[f] + wk * g)
                    return tuple(new)

                accs = lax.fori_loop(
                    0, K, kbody,
                    tuple(jnp.zeros((L,), jnp.float32) for _ in range(L)))
                for f in range(L):
                    out_buf[half * L + f, pl.ds(rr, L)] = accs[f] * (1.0 / K)
            return 0

        lax.fori_loop(0, NBLK, block, 0)
        pending_out[p] = pltpu.async_copy(
            out_buf, out_hbm.at[pl.ds(col0, FCW), pl.ds(row0, RPS)], sout[p])

    for cp in pending_out:
        if cp is not None:
            cp.wait()


@jax.jit
def _sc_call(feat_t, wt, nt):
    mesh = plsc.VectorSubcoreMesh(core_axis_name="c", subcore_axis_name="s")
    return pl.kernel(
        _body,
        out_type=jax.ShapeDtypeStruct((F, N_PAD), jnp.float32),
        mesh=mesh,
        compiler_params=pltpu.CompilerParams(use_tc_tiling_on_sc=False,
                                             needs_layout_passes=False),
        scratch_types=[
            pltpu.VMEM((FCW, N_UP), jnp.float32),
            pltpu.VMEM((K, RPS), jnp.int32),
            pltpu.VMEM((K, RPS), jnp.int32),
            pltpu.VMEM((K, RPS), jnp.float32),
            pltpu.VMEM((K, RPS), jnp.float32),
            pltpu.VMEM((FCW, RPS), jnp.float32),
            pltpu.VMEM((FCW, RPS), jnp.float32),
            pltpu.SemaphoreType.DMA,
            pltpu.SemaphoreType.DMA,
            pltpu.SemaphoreType.DMA,
            pltpu.SemaphoreType.DMA,
        ],
    )(feat_t, wt, nt)


def kernel(features, sel_idx_up, weights_down, nidx_down):
    n_down = weights_down.shape[0]
    pad = N_PAD - n_down
    wt = jnp.pad(weights_down, ((0, pad), (0, 0))).T
    nt = jnp.pad(nidx_down, ((0, pad), (0, 0))).T
    out_t = _sc_call(features.T, wt, nt)
    return out_t.T[:n_down]


# speedup vs baseline: 1.0080x; 1.0080x over previous
import jax
import jax.numpy as jnp
from jax import lax
from jax.experimental import pallas as pl
from jax.experimental.pallas import tpu as pltpu
from jax.experimental.pallas import tpu_sc as plsc

N_UP = 2500
F = 128
K = 32
N_PAD = 10240
L = 16

RG = 8
FC = 4
RPG = N_PAD // RG
SUB = 5
RPS = RPG // SUB
NBLK = RPS // L
FCW = F // FC


def _body(feat_hbm, wt_hbm, nt_hbm, out_hbm, feat_c,
          idx0, idx1, w0, w1, ob0, ob1, sin0, sin1, sout0, sout1):
    wid = lax.axis_index("s") * 2 + lax.axis_index("c")
    rg = wid // FC
    fc = wid % FC
    row0g = rg * RPG
    col0 = fc * FCW
    feat_cp = pltpu.async_copy(feat_hbm.at[pl.ds(col0, FCW)], feat_c, sout0)

    idx_b = (idx0, idx1)
    w_b = (w0, w1)
    out_b = (ob0, ob1)
    sin = (sin0, sin1)
    sout = (sout0, sout1)

    def stage(s, p):
        row0 = row0g + s * RPS
        return (pltpu.async_copy(nt_hbm.at[:, pl.ds(row0, RPS)],
                                 idx_b[p], sin[p]),
                pltpu.async_copy(wt_hbm.at[:, pl.ds(row0, RPS)],
                                 w_b[p], sin[p]))

    pending_in = stage(0, 0)
    feat_cp.wait()
    pending_out = [None, None]

    for s in range(SUB):
        p = s % 2
        row0 = row0g + s * RPS
        for cp in pending_in:
            cp.wait()
        if s + 1 < SUB:
            pending_in = stage(s + 1, (s + 1) % 2)
        if pending_out[p] is not None:
            pending_out[p].wait()
        idx_c = idx_b[p]
        w_c = w_b[p]
        out_buf = out_b[p]

        def block(b, _):
            rr = b * L
            for half in range(2):
                def kbody(k, accs):
                    vk = idx_c[k, pl.ds(rr, L)]
                    m = vk < N_UP
                    vkc = jnp.where(m, vk, 0)
                    wk = jnp.where(m, w_c[k, pl.ds(rr, L)], 0.0)
                    new = []
                    for f in range(L):
                        col = jnp.full((L,), half * L + f, jnp.int32)
                        g = plsc.load_gather(feat_c, [col, vkc])
                        new.append(accs[f] + wk * g)
                    return tuple(new)

                accs = lax.fori_loop(
                    0, K, kbody,
                    tuple(jnp.zeros((L,), jnp.float32) for _ in range(L)))
                for f in range(L):
                    out_buf[half * L + f, pl.ds(rr, L)] = accs[f] * (1.0 / K)
            return 0

        lax.fori_loop(0, NBLK, block, 0)
        pending_out[p] = pltpu.async_copy(
            out_buf, out_hbm.at[pl.ds(col0, FCW), pl.ds(row0, RPS)], sout[p])

    for cp in pending_out:
        if cp is not None:
            cp.wait()


@jax.jit
def _sc_call(feat_t, wt, nt):
    mesh = plsc.VectorSubcoreMesh(core_axis_name="c", subcore_axis_name="s")
    return pl.kernel(
        _body,
        out_type=jax.ShapeDtypeStruct((F, N_PAD), jnp.float32),
        mesh=mesh,
        compiler_params=pltpu.CompilerParams(use_tc_tiling_on_sc=False,
                                             needs_layout_passes=False),
        scratch_types=[
            pltpu.VMEM((FCW, N_UP), jnp.float32),
            pltpu.VMEM((K, RPS), jnp.int32),
            pltpu.VMEM((K, RPS), jnp.int32),
            pltpu.VMEM((K, RPS), jnp.float32),
            pltpu.VMEM((K, RPS), jnp.float32),
            pltpu.VMEM((FCW, RPS), jnp.float32),
            pltpu.VMEM((FCW, RPS), jnp.float32),
            pltpu.SemaphoreType.DMA,
            pltpu.SemaphoreType.DMA,
            pltpu.SemaphoreType.DMA,
            pltpu.SemaphoreType.DMA,
        ],
    )(feat_t, wt, nt)


def kernel(features, sel_idx_up, weights_down, nidx_down):
    n_down = weights_down.shape[0]
    pad = N_PAD - n_down
    wt = jnp.pad(weights_down, ((0, pad), (0, 0))).T
    nt = jnp.pad(nidx_down, ((0, pad), (0, 0))).T
    out_t = _sc_call(features.T, wt, nt)
    return out_t.T[:n_down]
